# Initial kernel scaffold; baseline (speedup 1.0000x reference)
import jax
import jax.numpy as jnp
from jax.experimental import pallas as pl


def kernel(x, labels, scores, Wq, Wk, Wv, Wp):
    raise NotImplementedError("write your pallas kernel here")


# speedup vs baseline: 1.0153x; 1.0153x over previous
import jax
import jax.numpy as jnp
from jax.experimental import pallas as pl

DIM = 1024
QK_DIM = 1024
HEADS = 16
GS = 64
WS = 2 * GS
NUM_CLUSTERS = 8
R_BASE, R_MIN, R_MAX = 0.5, 0.25, 0.75
LAMBDA_P, LAMBDA_V = 0.25, 0.25
CROSS_RATIO = 0.125
DQK = QK_DIM // HEADS
DV = DIM // HEADS


def _proj_kernel(x_ref, wq_ref, wk_ref, wv_ref, q_ref, k_ref, v_ref):
    x = x_ref[...]
    q_ref[...] = jnp.dot(x, wq_ref[...], preferred_element_type=jnp.float32)
    k_ref[...] = jnp.dot(x, wk_ref[...], preferred_element_type=jnp.float32)
    v_ref[...] = jnp.dot(x, wv_ref[...], preferred_element_type=jnp.float32)


def _attn_kernel(q_ref, k0_ref, k1_ref, v0_ref, v1_ref,
                 ql_ref, kl0_ref, kl1_ref, qs_ref, ks0_ref, ks1_ref,
                 wp_ref, out_ref):
    q = q_ref[...]
    k = jnp.concatenate([k0_ref[...], k1_ref[...]], axis=0)
    v = jnp.concatenate([v0_ref[...], v1_ref[...]], axis=0)
    ql = ql_ref[0]
    kl = jnp.concatenate([kl0_ref[0], kl1_ref[0]], axis=1)
    qs = qs_ref[0]
    ks = jnp.concatenate([ks0_ref[0], ks1_ref[0]], axis=1)

    cl = jax.lax.broadcasted_iota(jnp.int32, (NUM_CLUSTERS, GS), 0)
    counts = jnp.sum((ql == cl).astype(jnp.float32), axis=1, keepdims=True)
    maxc = jnp.max(counts)
    cid = jax.lax.broadcasted_iota(jnp.int32, (NUM_CLUSTERS, 1), 0)
    mode = jnp.min(jnp.where(counts == maxc, cid, NUM_CLUSTERS))
    purity = jnp.mean((ql == mode).astype(jnp.float32))
    smean = jnp.mean(qs)
    svar = jnp.mean((qs - smean) ** 2)
    focus = jnp.clip(R_BASE + LAMBDA_P * purity - LAMBDA_V * svar, R_MIN, R_MAX)
    keep = jnp.clip(jnp.ceil(focus * WS), 1.0, float(WS))
    cross_keep = jnp.round(keep * CROSS_RATIO)
    cross_keep = jnp.where(keep > 1.0, jnp.maximum(cross_keep, 1.0), 0.0)
    same_keep = jnp.maximum(keep - cross_keep, 1.0)

    same_mask = (ql[:, :, None] == kl[:, None, :])[0]
    conf = (qs[:, :, None] * ks[:, None, :])[0]
    jj = jax.lax.broadcasted_iota(jnp.int32, (WS, WS), 0)
    kk = jax.lax.broadcasted_iota(jnp.int32, (WS, WS), 1)
    tri = (kk < jj)[None, :, :]

    def topk_mask(vals, kn):
        vj = vals[:, :, None]
        vk = vals[:, None, :]
        hit = (vk > vj) | ((vk == vj) & tri)
        rank = jnp.sum(jnp.where(hit, 1.0, 0.0), axis=2)
        return rank < kn

    scale = DQK ** (-0.5)
    outs = []
    for h in range(HEADS):
        qh = q[:, h * DQK:(h + 1) * DQK]
        kh = k[:, h * DQK:(h + 1) * DQK]
        vh = v[:, h * DV:(h + 1) * DV]
        logits = jax.lax.dot_general(
            qh, kh, (((1,), (1,)), ((), ())),
            preferred_element_type=jnp.float32) * scale
        m = jnp.max(logits, axis=1, keepdims=True)
        e = jnp.exp(logits - m)
        a = e / jnp.sum(e, axis=1, keepdims=True)

        same_scores = jnp.where(same_mask, a, -10000.0)
        cross_scores = jnp.where(same_mask, -10000.0, a * conf)
        fm = ((topk_mask(same_scores, same_keep) & same_mask) |
              (topk_mask(cross_scores, cross_keep) & ~same_mask))
        att = jnp.where(fm, a, 0.0)
        att = att / (jnp.sum(att, axis=1, keepdims=True) + 1e-9)
        outs.append(jax.lax.dot_general(
            att, vh, (((1,), (0,)), ((), ())),
            preferred_element_type=jnp.float32))
    o = jnp.concatenate(outs, axis=1)
    out_ref[...] = jnp.dot(o, wp_ref[...], preferred_element_type=jnp.float32)


def kernel(x, labels, scores, Wq, Wk, Wv, Wp):
    B, N, _ = x.shape
    ng = N // GS
    x0, lab0, sc0 = x[0], labels[0], scores[0]
    x_ext = jnp.concatenate([x0, x0[N - GS:][::-1]], axis=0)
    lab_ext = jnp.concatenate([lab0, lab0[N - GS:][::-1]], axis=0)
    sc_ext = jnp.concatenate([sc0, sc0[N - GS:][::-1]], axis=0)

    R = 256
    n_ext = N + GS
    n_pad = ((n_ext + R - 1) // R) * R
    x_ext = jnp.pad(x_ext, ((0, n_pad - n_ext), (0, 0)))

    q_ext, k_ext, v_ext = pl.pallas_call(
        _proj_kernel,
        grid=(n_pad // R,),
        in_specs=[
            pl.BlockSpec((R, DIM), lambda i: (i, 0)),
            pl.BlockSpec((DIM, QK_DIM), lambda i: (0, 0)),
            pl.BlockSpec((DIM, QK_DIM), lambda i: (0, 0)),
            pl.BlockSpec((DIM, DIM), lambda i: (0, 0)),
        ],
        out_specs=[
            pl.BlockSpec((R, QK_DIM), lambda i: (i, 0)),
            pl.BlockSpec((R, QK_DIM), lambda i: (i, 0)),
            pl.BlockSpec((R, DIM), lambda i: (i, 0)),
        ],
        out_shape=[
            jax.ShapeDtypeStruct((n_pad, QK_DIM), jnp.float32),
            jax.ShapeDtypeStruct((n_pad, QK_DIM), jnp.float32),
            jax.ShapeDtypeStruct((n_pad, DIM), jnp.float32),
        ],
    )(x_ext, Wq, Wk, Wv)

    lab3 = lab_ext.reshape(n_ext // GS, 1, GS)
    sc3 = sc_ext.reshape(n_ext // GS, 1, GS)

    out = pl.pallas_call(
        _attn_kernel,
        grid=(ng,),
        in_specs=[
            pl.BlockSpec((GS, QK_DIM), lambda i: (i, 0)),
            pl.BlockSpec((GS, QK_DIM), lambda i: (i, 0)),
            pl.BlockSpec((GS, QK_DIM), lambda i: (i + 1, 0)),
            pl.BlockSpec((GS, DIM), lambda i: (i, 0)),
            pl.BlockSpec((GS, DIM), lambda i: (i + 1, 0)),
            pl.BlockSpec((1, 1, GS), lambda i: (i, 0, 0)),
            pl.BlockSpec((1, 1, GS), lambda i: (i, 0, 0)),
            pl.BlockSpec((1, 1, GS), lambda i: (i + 1, 0, 0)),
            pl.BlockSpec((1, 1, GS), lambda i: (i, 0, 0)),
            pl.BlockSpec((1, 1, GS), lambda i: (i, 0, 0)),
            pl.BlockSpec((1, 1, GS), lambda i: (i + 1, 0, 0)),
            pl.BlockSpec((DIM, DIM), lambda i: (0, 0)),
        ],
        out_specs=pl.BlockSpec((GS, DIM), lambda i: (i, 0)),
        out_shape=jax.ShapeDtypeStruct((N, DIM), jnp.float32),
    )(q_ext, k_ext, k_ext, v_ext, v_ext,
      lab3, lab3, lab3, sc3, sc3, sc3, Wp)

    return out[None]


# speedup vs baseline: 3.3010x; 3.2512x over previous
import jax
import jax.numpy as jnp
from jax.experimental import pallas as pl
from jax.experimental.pallas import tpu as pltpu

DIM = 1024
QK_DIM = 1024
HEADS = 16
GS = 64
WS = 2 * GS
NUM_CLUSTERS = 8
R_BASE, R_MIN, R_MAX = 0.5, 0.25, 0.75
LAMBDA_P, LAMBDA_V = 0.25, 0.25
CROSS_RATIO = 0.125
DQK = QK_DIM // HEADS
DV = DIM // HEADS


def _proj_kernel(x_ref, wq_ref, wk_ref, wv_ref, q_ref, k_ref, v_ref):
    x = x_ref[...]
    q_ref[...] = jnp.dot(x, wq_ref[...], preferred_element_type=jnp.float32)
    k_ref[...] = jnp.dot(x, wk_ref[...], preferred_element_type=jnp.float32)
    v_ref[...] = jnp.dot(x, wv_ref[...], preferred_element_type=jnp.float32)


def _attn_kernel(q_ref, k0_ref, k1_ref, v0_ref, v1_ref,
                 ql_ref, kl0_ref, kl1_ref, qs_ref, ks0_ref, ks1_ref,
                 wp_ref, out_ref):
    q = q_ref[...]
    k = jnp.concatenate([k0_ref[...], k1_ref[...]], axis=0)
    v = jnp.concatenate([v0_ref[...], v1_ref[...]], axis=0)
    ql = ql_ref[0]
    kl = jnp.concatenate([kl0_ref[0], kl1_ref[0]], axis=1)
    qs = qs_ref[0]
    ks = jnp.concatenate([ks0_ref[0], ks1_ref[0]], axis=1)

    cl = jax.lax.broadcasted_iota(jnp.int32, (NUM_CLUSTERS, GS), 0)
    counts = jnp.sum((ql == cl).astype(jnp.float32), axis=1, keepdims=True)
    maxc = jnp.max(counts)
    cid = jax.lax.broadcasted_iota(jnp.int32, (NUM_CLUSTERS, 1), 0)
    mode = jnp.min(jnp.where(counts == maxc, cid, NUM_CLUSTERS))
    purity = jnp.mean((ql == mode).astype(jnp.float32))
    smean = jnp.mean(qs)
    svar = jnp.mean((qs - smean) ** 2)
    focus = jnp.clip(R_BASE + LAMBDA_P * purity - LAMBDA_V * svar, R_MIN, R_MAX)
    keep = jnp.clip(jnp.ceil(focus * WS), 1.0, float(WS))
    cross_keep = jnp.round(keep * CROSS_RATIO)
    cross_keep = jnp.where(keep > 1.0, jnp.maximum(cross_keep, 1.0), 0.0)
    same_keep = jnp.maximum(keep - cross_keep, 1.0)

    same_mask = (ql[:, :, None] == kl[:, None, :])[0]
    conf = (qs[:, :, None] * ks[:, None, :])[0]
    lane = jax.lax.broadcasted_iota(jnp.int32, (1, WS), 1)

    def sort_desc(x):
        for s_exp in range(1, 8):
            s = 1 << s_exp
            for d_exp in range(s_exp - 1, -1, -1):
                d = 1 << d_exp
                has_d = (lane & d) != 0
                partner = jnp.where(has_d, pltpu.roll(x, d, axis=1),
                                    pltpu.roll(x, WS - d, axis=1))
                take_max = ((lane & s) == 0) ^ has_d
                x = jnp.where(take_max, jnp.maximum(x, partner),
                              jnp.minimum(x, partner))
        return x

    def topk_mask(vals, kn):
        srt = sort_desc(vals)
        kn_i = kn.astype(jnp.int32)
        t = jnp.sum(jnp.where(lane == kn_i - 1, srt, 0.0),
                    axis=1, keepdims=True)
        gt = vals > t
        n_gt = jnp.sum(gt.astype(jnp.float32), axis=1, keepdims=True)
        eq = vals == t
        run = eq.astype(jnp.float32)
        inc = run
        for sh in (1, 2, 4, 8, 16, 32, 64):
            inc = inc + jnp.where(lane >= sh, pltpu.roll(inc, sh, axis=1), 0.0)
        eq_lower = inc - run
        return gt | (eq & (eq_lower < kn - n_gt))

    scale = DQK ** (-0.5)
    outs = []
    for h in range(HEADS):
        qh = q[:, h * DQK:(h + 1) * DQK]
        kh = k[:, h * DQK:(h + 1) * DQK]
        vh = v[:, h * DV:(h + 1) * DV]
        logits = jax.lax.dot_general(
            qh, kh, (((1,), (1,)), ((), ())),
            preferred_element_type=jnp.float32) * scale
        m = jnp.max(logits, axis=1, keepdims=True)
        e = jnp.exp(logits - m)
        a = e / jnp.sum(e, axis=1, keepdims=True)

        same_scores = jnp.where(same_mask, a, -10000.0)
        cross_scores = jnp.where(same_mask, -10000.0, a * conf)
        fm = ((topk_mask(same_scores, same_keep) & same_mask) |
              (topk_mask(cross_scores, cross_keep) & ~same_mask))
        att = jnp.where(fm, a, 0.0)
        att = att / (jnp.sum(att, axis=1, keepdims=True) + 1e-9)
        outs.append(jax.lax.dot_general(
            att, vh, (((1,), (0,)), ((), ())),
            preferred_element_type=jnp.float32))
    o = jnp.concatenate(outs, axis=1)
    out_ref[...] = jnp.dot(o, wp_ref[...], preferred_element_type=jnp.float32)


def kernel(x, labels, scores, Wq, Wk, Wv, Wp):
    B, N, _ = x.shape
    ng = N // GS
    x0, lab0, sc0 = x[0], labels[0], scores[0]
    x_ext = jnp.concatenate([x0, x0[N - GS:][::-1]], axis=0)
    lab_ext = jnp.concatenate([lab0, lab0[N - GS:][::-1]], axis=0)
    sc_ext = jnp.concatenate([sc0, sc0[N - GS:][::-1]], axis=0)

    R = 256
    n_ext = N + GS
    n_pad = ((n_ext + R - 1) // R) * R
    x_ext = jnp.pad(x_ext, ((0, n_pad - n_ext), (0, 0)))

    q_ext, k_ext, v_ext = pl.pallas_call(
        _proj_kernel,
        grid=(n_pad // R,),
        in_specs=[
            pl.BlockSpec((R, DIM), lambda i: (i, 0)),
            pl.BlockSpec((DIM, QK_DIM), lambda i: (0, 0)),
            pl.BlockSpec((DIM, QK_DIM), lambda i: (0, 0)),
            pl.BlockSpec((DIM, DIM), lambda i: (0, 0)),
        ],
        out_specs=[
            pl.BlockSpec((R, QK_DIM), lambda i: (i, 0)),
            pl.BlockSpec((R, QK_DIM), lambda i: (i, 0)),
            pl.BlockSpec((R, DIM), lambda i: (i, 0)),
        ],
        out_shape=[
            jax.ShapeDtypeStruct((n_pad, QK_DIM), jnp.float32),
            jax.ShapeDtypeStruct((n_pad, QK_DIM), jnp.float32),
            jax.ShapeDtypeStruct((n_pad, DIM), jnp.float32),
        ],
    )(x_ext, Wq, Wk, Wv)

    lab3 = lab_ext.reshape(n_ext // GS, 1, GS)
    sc3 = sc_ext.reshape(n_ext // GS, 1, GS)

    out = pl.pallas_call(
        _attn_kernel,
        grid=(ng,),
        in_specs=[
            pl.BlockSpec((GS, QK_DIM), lambda i: (i, 0)),
            pl.BlockSpec((GS, QK_DIM), lambda i: (i, 0)),
            pl.BlockSpec((GS, QK_DIM), lambda i: (i + 1, 0)),
            pl.BlockSpec((GS, DIM), lambda i: (i, 0)),
            pl.BlockSpec((GS, DIM), lambda i: (i + 1, 0)),
            pl.BlockSpec((1, 1, GS), lambda i: (i, 0, 0)),
            pl.BlockSpec((1, 1, GS), lambda i: (i, 0, 0)),
            pl.BlockSpec((1, 1, GS), lambda i: (i + 1, 0, 0)),
            pl.BlockSpec((1, 1, GS), lambda i: (i, 0, 0)),
            pl.BlockSpec((1, 1, GS), lambda i: (i, 0, 0)),
            pl.BlockSpec((1, 1, GS), lambda i: (i + 1, 0, 0)),
            pl.BlockSpec((DIM, DIM), lambda i: (0, 0)),
        ],
        out_specs=pl.BlockSpec((GS, DIM), lambda i: (i, 0)),
        out_shape=jax.ShapeDtypeStruct((N, DIM), jnp.float32),
    )(q_ext, k_ext, k_ext, v_ext, v_ext,
      lab3, lab3, lab3, sc3, sc3, sc3, Wp)

    return out[None]


# speedup vs baseline: 3.3059x; 1.0015x over previous
import jax
import jax.numpy as jnp
from jax.experimental import pallas as pl
from jax.experimental.pallas import tpu as pltpu

DIM = 1024
QK_DIM = 1024
HEADS = 16
GS = 64
WS = 2 * GS
NUM_CLUSTERS = 8
R_BASE, R_MIN, R_MAX = 0.5, 0.25, 0.75
LAMBDA_P, LAMBDA_V = 0.25, 0.25
CROSS_RATIO = 0.125
DQK = QK_DIM // HEADS
DV = DIM // HEADS


def _proj_kernel(x_ref, wq_ref, wk_ref, wv_ref, q_ref, k_ref, v_ref):
    x = x_ref[...]
    q_ref[...] = jnp.dot(x, wq_ref[...], preferred_element_type=jnp.float32)
    k_ref[...] = jnp.dot(x, wk_ref[...], preferred_element_type=jnp.float32)
    v_ref[...] = jnp.dot(x, wv_ref[...], preferred_element_type=jnp.float32)


def _attn_kernel(q_ref, k0_ref, k1_ref, v0_ref, v1_ref,
                 ql_ref, kl0_ref, kl1_ref, qs_ref, ks0_ref, ks1_ref,
                 wp_ref, out_ref):
    q = q_ref[...]
    k = jnp.concatenate([k0_ref[...], k1_ref[...]], axis=0)
    v = jnp.concatenate([v0_ref[...], v1_ref[...]], axis=0)
    ql = ql_ref[0]
    kl = jnp.concatenate([kl0_ref[0], kl1_ref[0]], axis=1)
    qs = qs_ref[0]
    ks = jnp.concatenate([ks0_ref[0], ks1_ref[0]], axis=1)

    cl = jax.lax.broadcasted_iota(jnp.int32, (NUM_CLUSTERS, GS), 0)
    counts = jnp.sum((ql == cl).astype(jnp.float32), axis=1, keepdims=True)
    maxc = jnp.max(counts)
    cid = jax.lax.broadcasted_iota(jnp.int32, (NUM_CLUSTERS, 1), 0)
    mode = jnp.min(jnp.where(counts == maxc, cid, NUM_CLUSTERS))
    purity = jnp.mean((ql == mode).astype(jnp.float32))
    smean = jnp.mean(qs)
    svar = jnp.mean((qs - smean) ** 2)
    focus = jnp.clip(R_BASE + LAMBDA_P * purity - LAMBDA_V * svar, R_MIN, R_MAX)
    keep = jnp.clip(jnp.ceil(focus * WS), 1.0, float(WS))
    cross_keep = jnp.round(keep * CROSS_RATIO)
    cross_keep = jnp.where(keep > 1.0, jnp.maximum(cross_keep, 1.0), 0.0)
    same_keep = jnp.maximum(keep - cross_keep, 1.0)

    same_mask = (ql[:, :, None] == kl[:, None, :])[0]
    conf = (qs[:, :, None] * ks[:, None, :])[0]
    lane = jax.lax.broadcasted_iota(jnp.int32, (1, WS), 1)
    n_same = jnp.sum(same_mask.astype(jnp.float32), axis=1, keepdims=True)

    def sort_desc(x):
        for s_exp in range(1, 8):
            s = 1 << s_exp
            for d_exp in range(s_exp - 1, -1, -1):
                d = 1 << d_exp
                has_d = (lane & d) != 0
                partner = jnp.where(has_d, pltpu.roll(x, d, axis=1),
                                    pltpu.roll(x, WS - d, axis=1))
                take_max = ((lane & s) == 0) ^ has_d
                x = jnp.where(take_max, jnp.maximum(x, partner),
                              jnp.minimum(x, partner))
        return x

    def rank_mask(enc, srt, kn):
        idx = jnp.minimum(kn, float(WS)).astype(jnp.int32) - 1
        t = jnp.sum(jnp.where(lane == idx, srt, 0.0),
                    axis=1, keepdims=True)
        gt = enc > t
        n_gt = jnp.sum(gt.astype(jnp.float32), axis=1, keepdims=True)
        eq = enc == t
        run = eq.astype(jnp.float32)
        inc = run
        for sh in (1, 2, 4, 8, 16, 32, 64):
            inc = inc + jnp.where(lane >= sh, pltpu.roll(inc, sh, axis=1), 0.0)
        eq_lower = inc - run
        return gt | (eq & (eq_lower < kn - n_gt))

    scale = DQK ** (-0.5)
    outs = []
    for h in range(HEADS):
        qh = q[:, h * DQK:(h + 1) * DQK]
        kh = k[:, h * DQK:(h + 1) * DQK]
        vh = v[:, h * DV:(h + 1) * DV]
        logits = jax.lax.dot_general(
            qh, kh, (((1,), (1,)), ((), ())),
            preferred_element_type=jnp.float32) * scale
        m = jnp.max(logits, axis=1, keepdims=True)
        e = jnp.exp(logits - m)
        a = e / jnp.sum(e, axis=1, keepdims=True)

        enc = jnp.where(same_mask, a + 2.0, a * conf)
        srt = sort_desc(enc)
        fm = ((rank_mask(enc, srt, same_keep) & same_mask) |
              (rank_mask(enc, srt, n_same + cross_keep) & ~same_mask))
        att = jnp.where(fm, a, 0.0)
        att = att / (jnp.sum(att, axis=1, keepdims=True) + 1e-9)
        outs.append(jax.lax.dot_general(
            att, vh, (((1,), (0,)), ((), ())),
            preferred_element_type=jnp.float32))
    o = jnp.concatenate(outs, axis=1)
    out_ref[...] = jnp.dot(o, wp_ref[...], preferred_element_type=jnp.float32)


def kernel(x, labels, scores, Wq, Wk, Wv, Wp):
    B, N, _ = x.shape
    ng = N // GS
    x0, lab0, sc0 = x[0], labels[0], scores[0]
    x_ext = jnp.concatenate([x0, x0[N - GS:][::-1]], axis=0)
    lab_ext = jnp.concatenate([lab0, lab0[N - GS:][::-1]], axis=0)
    sc_ext = jnp.concatenate([sc0, sc0[N - GS:][::-1]], axis=0)

    R = 256
    n_ext = N + GS
    n_pad = ((n_ext + R - 1) // R) * R
    x_ext = jnp.pad(x_ext, ((0, n_pad - n_ext), (0, 0)))

    q_ext, k_ext, v_ext = pl.pallas_call(
        _proj_kernel,
        grid=(n_pad // R,),
        in_specs=[
            pl.BlockSpec((R, DIM), lambda i: (i, 0)),
            pl.BlockSpec((DIM, QK_DIM), lambda i: (0, 0)),
            pl.BlockSpec((DIM, QK_DIM), lambda i: (0, 0)),
            pl.BlockSpec((DIM, DIM), lambda i: (0, 0)),
        ],
        out_specs=[
            pl.BlockSpec((R, QK_DIM), lambda i: (i, 0)),
            pl.BlockSpec((R, QK_DIM), lambda i: (i, 0)),
            pl.BlockSpec((R, DIM), lambda i: (i, 0)),
        ],
        out_shape=[
            jax.ShapeDtypeStruct((n_pad, QK_DIM), jnp.float32),
            jax.ShapeDtypeStruct((n_pad, QK_DIM), jnp.float32),
            jax.ShapeDtypeStruct((n_pad, DIM), jnp.float32),
        ],
    )(x_ext, Wq, Wk, Wv)

    lab3 = lab_ext.reshape(n_ext // GS, 1, GS)
    sc3 = sc_ext.reshape(n_ext // GS, 1, GS)

    out = pl.pallas_call(
        _attn_kernel,
        grid=(ng,),
        in_specs=[
            pl.BlockSpec((GS, QK_DIM), lambda i: (i, 0)),
            pl.BlockSpec((GS, QK_DIM), lambda i: (i, 0)),
            pl.BlockSpec((GS, QK_DIM), lambda i: (i + 1, 0)),
            pl.BlockSpec((GS, DIM), lambda i: (i, 0)),
            pl.BlockSpec((GS, DIM), lambda i: (i + 1, 0)),
            pl.BlockSpec((1, 1, GS), lambda i: (i, 0, 0)),
            pl.BlockSpec((1, 1, GS), lambda i: (i, 0, 0)),
            pl.BlockSpec((1, 1, GS), lambda i: (i + 1, 0, 0)),
            pl.BlockSpec((1, 1, GS), lambda i: (i, 0, 0)),
            pl.BlockSpec((1, 1, GS), lambda i: (i, 0, 0)),
            pl.BlockSpec((1, 1, GS), lambda i: (i + 1, 0, 0)),
            pl.BlockSpec((DIM, DIM), lambda i: (0, 0)),
        ],
        out_specs=pl.BlockSpec((GS, DIM), lambda i: (i, 0)),
        out_shape=jax.ShapeDtypeStruct((N, DIM), jnp.float32),
    )(q_ext, k_ext, k_ext, v_ext, v_ext,
      lab3, lab3, lab3, sc3, sc3, sc3, Wp)

    return out[None]


# speedup vs baseline: 10.8868x; 3.2932x over previous
import jax
import jax.numpy as jnp
from jax.experimental import pallas as pl
from jax.experimental.pallas import tpu as pltpu

DIM = 1024
QK_DIM = 1024
HEADS = 16
GS = 64
WS = 2 * GS
NUM_CLUSTERS = 8
R_BASE, R_MIN, R_MAX = 0.5, 0.25, 0.75
LAMBDA_P, LAMBDA_V = 0.25, 0.25
CROSS_RATIO = 0.125
DQK = QK_DIM // HEADS
DV = DIM // HEADS


def _proj_kernel(x_ref, wq_ref, wk_ref, wv_ref, q_ref, k_ref, v_ref):
    x = x_ref[...]
    q_ref[...] = jnp.dot(x, wq_ref[...], preferred_element_type=jnp.float32)
    k_ref[...] = jnp.dot(x, wk_ref[...], preferred_element_type=jnp.float32)
    v_ref[...] = jnp.dot(x, wv_ref[...], preferred_element_type=jnp.float32)


def _attn_kernel(q_ref, k0_ref, k1_ref, v0_ref, v1_ref,
                 ql_ref, kl0_ref, kl1_ref, qs_ref, ks0_ref, ks1_ref,
                 wp_ref, out_ref):
    q = q_ref[...]
    k = jnp.concatenate([k0_ref[...], k1_ref[...]], axis=0)
    v = jnp.concatenate([v0_ref[...], v1_ref[...]], axis=0)
    ql = ql_ref[0]
    kl = jnp.concatenate([kl0_ref[0], kl1_ref[0]], axis=1)
    qs = qs_ref[0]
    ks = jnp.concatenate([ks0_ref[0], ks1_ref[0]], axis=1)

    cl = jax.lax.broadcasted_iota(jnp.int32, (NUM_CLUSTERS, GS), 0)
    counts = jnp.sum((ql == cl).astype(jnp.float32), axis=1, keepdims=True)
    maxc = jnp.max(counts)
    cid = jax.lax.broadcasted_iota(jnp.int32, (NUM_CLUSTERS, 1), 0)
    mode = jnp.min(jnp.where(counts == maxc, cid, NUM_CLUSTERS))
    purity = jnp.mean((ql == mode).astype(jnp.float32))
    smean = jnp.mean(qs)
    svar = jnp.mean((qs - smean) ** 2)
    focus = jnp.clip(R_BASE + LAMBDA_P * purity - LAMBDA_V * svar, R_MIN, R_MAX)
    keep = jnp.clip(jnp.ceil(focus * WS), 1.0, float(WS))
    cross_keep = jnp.round(keep * CROSS_RATIO)
    cross_keep = jnp.where(keep > 1.0, jnp.maximum(cross_keep, 1.0), 0.0)
    same_keep = jnp.maximum(keep - cross_keep, 1.0)

    same_mask = (ql[:, :, None] == kl[:, None, :])[0]
    conf = (qs[:, :, None] * ks[:, None, :])[0]
    lane = jax.lax.broadcasted_iota(jnp.int32, (1, WS), 1)
    n_same = jnp.sum(same_mask.astype(jnp.float32), axis=1, keepdims=True)

    def sort_desc(x):
        for s_exp in range(1, 8):
            s = 1 << s_exp
            for d_exp in range(s_exp - 1, -1, -1):
                d = 1 << d_exp
                has_d = (lane & d) != 0
                partner = jnp.where(has_d, pltpu.roll(x, d, axis=1),
                                    pltpu.roll(x, WS - d, axis=1))
                take_max = ((lane & s) == 0) ^ has_d
                x = jnp.where(take_max, jnp.maximum(x, partner),
                              jnp.minimum(x, partner))
        return x

    def rank_mask(enc, srt, kn):
        idx = jnp.minimum(kn, float(WS)).astype(jnp.int32) - 1
        t = jnp.sum(jnp.where(lane == idx, srt, 0.0),
                    axis=1, keepdims=True)
        gt = enc > t
        n_gt = jnp.sum(gt.astype(jnp.float32), axis=1, keepdims=True)
        eq = enc == t
        run = eq.astype(jnp.float32)
        inc = run
        for sh in (1, 2, 4, 8, 16, 32, 64):
            inc = inc + jnp.where(lane >= sh, pltpu.roll(inc, sh, axis=1), 0.0)
        eq_lower = inc - run
        return gt | (eq & (eq_lower < kn - n_gt))

    scale = DQK ** (-0.5)
    logits = [jax.lax.dot_general(
        q[:, h * DQK:(h + 1) * DQK], k[:, h * DQK:(h + 1) * DQK],
        (((1,), (1,)), ((), ())),
        preferred_element_type=jnp.float32) for h in range(HEADS)]
    lg = jnp.concatenate(logits, axis=0) * scale
    m = jnp.max(lg, axis=1, keepdims=True)
    e = jnp.exp(lg - m)
    a = e / jnp.sum(e, axis=1, keepdims=True)

    same_t = jnp.tile(same_mask, (HEADS, 1))
    conf_t = jnp.tile(conf, (HEADS, 1))
    kn_cross = jnp.tile(n_same + cross_keep, (HEADS, 1))

    enc = jnp.where(same_t, a + 2.0, a * conf_t)
    srt = sort_desc(enc)
    fm = ((rank_mask(enc, srt, same_keep) & same_t) |
          (rank_mask(enc, srt, kn_cross) & ~same_t))
    att = jnp.where(fm, a, 0.0)
    att = att / (jnp.sum(att, axis=1, keepdims=True) + 1e-9)

    outs = [jax.lax.dot_general(
        att[h * GS:(h + 1) * GS], v[:, h * DV:(h + 1) * DV],
        (((1,), (0,)), ((), ())),
        preferred_element_type=jnp.float32) for h in range(HEADS)]
    o = jnp.concatenate(outs, axis=1)
    out_ref[...] = jnp.dot(o, wp_ref[...], preferred_element_type=jnp.float32)


def kernel(x, labels, scores, Wq, Wk, Wv, Wp):
    B, N, _ = x.shape
    ng = N // GS
    x0, lab0, sc0 = x[0], labels[0], scores[0]
    x_ext = jnp.concatenate([x0, x0[N - GS:][::-1]], axis=0)
    lab_ext = jnp.concatenate([lab0, lab0[N - GS:][::-1]], axis=0)
    sc_ext = jnp.concatenate([sc0, sc0[N - GS:][::-1]], axis=0)

    R = 256
    n_ext = N + GS
    n_pad = ((n_ext + R - 1) // R) * R
    x_ext = jnp.pad(x_ext, ((0, n_pad - n_ext), (0, 0)))

    q_ext, k_ext, v_ext = pl.pallas_call(
        _proj_kernel,
        grid=(n_pad // R,),
        in_specs=[
            pl.BlockSpec((R, DIM), lambda i: (i, 0)),
            pl.BlockSpec((DIM, QK_DIM), lambda i: (0, 0)),
            pl.BlockSpec((DIM, QK_DIM), lambda i: (0, 0)),
            pl.BlockSpec((DIM, DIM), lambda i: (0, 0)),
        ],
        out_specs=[
            pl.BlockSpec((R, QK_DIM), lambda i: (i, 0)),
            pl.BlockSpec((R, QK_DIM), lambda i: (i, 0)),
            pl.BlockSpec((R, DIM), lambda i: (i, 0)),
        ],
        out_shape=[
            jax.ShapeDtypeStruct((n_pad, QK_DIM), jnp.float32),
            jax.ShapeDtypeStruct((n_pad, QK_DIM), jnp.float32),
            jax.ShapeDtypeStruct((n_pad, DIM), jnp.float32),
        ],
    )(x_ext, Wq, Wk, Wv)

    lab3 = lab_ext.reshape(n_ext // GS, 1, GS)
    sc3 = sc_ext.reshape(n_ext // GS, 1, GS)

    out = pl.pallas_call(
        _attn_kernel,
        grid=(ng,),
        in_specs=[
            pl.BlockSpec((GS, QK_DIM), lambda i: (i, 0)),
            pl.BlockSpec((GS, QK_DIM), lambda i: (i, 0)),
            pl.BlockSpec((GS, QK_DIM), lambda i: (i + 1, 0)),
            pl.BlockSpec((GS, DIM), lambda i: (i, 0)),
            pl.BlockSpec((GS, DIM), lambda i: (i + 1, 0)),
            pl.BlockSpec((1, 1, GS), lambda i: (i, 0, 0)),
            pl.BlockSpec((1, 1, GS), lambda i: (i, 0, 0)),
            pl.BlockSpec((1, 1, GS), lambda i: (i + 1, 0, 0)),
            pl.BlockSpec((1, 1, GS), lambda i: (i, 0, 0)),
            pl.BlockSpec((1, 1, GS), lambda i: (i, 0, 0)),
            pl.BlockSpec((1, 1, GS), lambda i: (i + 1, 0, 0)),
            pl.BlockSpec((DIM, DIM), lambda i: (0, 0)),
        ],
        out_specs=pl.BlockSpec((GS, DIM), lambda i: (i, 0)),
        out_shape=jax.ShapeDtypeStruct((N, DIM), jnp.float32),
    )(q_ext, k_ext, k_ext, v_ext, v_ext,
      lab3, lab3, lab3, sc3, sc3, sc3, Wp)

    return out[None]


# speedup vs baseline: 13.5372x; 1.2434x over previous
import jax
import jax.numpy as jnp
from jax.experimental import pallas as pl
from jax.experimental.pallas import tpu as pltpu

DIM = 1024
QK_DIM = 1024
HEADS = 16
GS = 64
WS = 2 * GS
NUM_CLUSTERS = 8
R_BASE, R_MIN, R_MAX = 0.5, 0.25, 0.75
LAMBDA_P, LAMBDA_V = 0.25, 0.25
CROSS_RATIO = 0.125
DQK = QK_DIM // HEADS
DV = DIM // HEADS


def _proj_kernel(x_ref, wq_ref, wk_ref, wv_ref, q_ref, k_ref, v_ref):
    x = x_ref[...]
    q_ref[...] = jnp.dot(x, wq_ref[...], preferred_element_type=jnp.float32)
    k_ref[...] = jnp.dot(x, wk_ref[...], preferred_element_type=jnp.float32)
    v_ref[...] = jnp.dot(x, wv_ref[...], preferred_element_type=jnp.float32)


def _attn_kernel(q_ref, k0_ref, k1_ref, v0_ref, v1_ref,
                 ql_ref, kl0_ref, kl1_ref, qs_ref, ks0_ref, ks1_ref,
                 wp_ref, out_ref):
    q = q_ref[...]
    k = jnp.concatenate([k0_ref[...], k1_ref[...]], axis=0)
    v = jnp.concatenate([v0_ref[...], v1_ref[...]], axis=0)
    ql = ql_ref[0]
    kl = jnp.concatenate([kl0_ref[0], kl1_ref[0]], axis=1)
    qs = qs_ref[0]
    ks = jnp.concatenate([ks0_ref[0], ks1_ref[0]], axis=1)

    cl = jax.lax.broadcasted_iota(jnp.int32, (NUM_CLUSTERS, GS), 0)
    counts = jnp.sum((ql == cl).astype(jnp.float32), axis=1, keepdims=True)
    maxc = jnp.max(counts)
    cid = jax.lax.broadcasted_iota(jnp.int32, (NUM_CLUSTERS, 1), 0)
    mode = jnp.min(jnp.where(counts == maxc, cid, NUM_CLUSTERS))
    purity = jnp.mean((ql == mode).astype(jnp.float32))
    smean = jnp.mean(qs)
    svar = jnp.mean((qs - smean) ** 2)
    focus = jnp.clip(R_BASE + LAMBDA_P * purity - LAMBDA_V * svar, R_MIN, R_MAX)
    keep = jnp.clip(jnp.ceil(focus * WS), 1.0, float(WS))
    cross_keep = jnp.round(keep * CROSS_RATIO)
    cross_keep = jnp.where(keep > 1.0, jnp.maximum(cross_keep, 1.0), 0.0)
    same_keep = jnp.maximum(keep - cross_keep, 1.0)

    same_mask = (ql[:, :, None] == kl[:, None, :])[0]
    conf = (qs[:, :, None] * ks[:, None, :])[0]
    lane = jax.lax.broadcasted_iota(jnp.int32, (1, WS), 1)
    n_same = jnp.sum(same_mask.astype(jnp.float32), axis=1, keepdims=True)

    def sort_desc(x):
        for s_exp in range(1, 8):
            s = 1 << s_exp
            for d_exp in range(s_exp - 1, -1, -1):
                d = 1 << d_exp
                has_d = (lane & d) != 0
                partner = jnp.where(has_d, pltpu.roll(x, d, axis=1),
                                    pltpu.roll(x, WS - d, axis=1))
                take_max = ((lane & s) == 0) ^ has_d
                x = jnp.where(take_max, jnp.maximum(x, partner),
                              jnp.minimum(x, partner))
        return x

    kk2 = jax.lax.broadcasted_iota(jnp.int32, (WS, WS), 0)
    jj2 = jax.lax.broadcasted_iota(jnp.int32, (WS, WS), 1)
    ones_m = jnp.ones((WS, WS), dtype=jnp.bfloat16)
    sl_m = (kk2 < jj2).astype(jnp.bfloat16)

    def rank_mask(enc, srt, kn):
        idx = jnp.minimum(kn, float(WS)).astype(jnp.int32) - 1
        t = jnp.sum(jnp.where(lane == idx, srt, 0.0),
                    axis=1, keepdims=True)
        gt = enc > t
        eq = enc == t
        cnt = (jax.lax.dot_general(
                   gt.astype(jnp.bfloat16), ones_m,
                   (((1,), (0,)), ((), ())),
                   preferred_element_type=jnp.float32) +
               jax.lax.dot_general(
                   eq.astype(jnp.bfloat16), sl_m,
                   (((1,), (0,)), ((), ())),
                   preferred_element_type=jnp.float32))
        return gt | (eq & (cnt < kn))

    scale = DQK ** (-0.5)
    logits = [jax.lax.dot_general(
        q[:, h * DQK:(h + 1) * DQK], k[:, h * DQK:(h + 1) * DQK],
        (((1,), (1,)), ((), ())),
        preferred_element_type=jnp.float32) for h in range(HEADS)]
    lg = jnp.concatenate(logits, axis=0) * scale
    m = jnp.max(lg, axis=1, keepdims=True)
    e = jnp.exp(lg - m)

    same_t = jnp.tile(same_mask, (HEADS, 1))
    conf_t = jnp.tile(conf, (HEADS, 1))
    kn_cross = jnp.tile(n_same + cross_keep, (HEADS, 1))

    enc = jnp.where(same_t, e + 2.0, e * conf_t)
    srt = sort_desc(enc)
    fm = ((rank_mask(enc, srt, same_keep) & same_t) |
          (rank_mask(enc, srt, kn_cross) & ~same_t))
    att = jnp.where(fm, e, 0.0)
    att = att / (jnp.sum(att, axis=1, keepdims=True) + 1e-9)

    outs = [jax.lax.dot_general(
        att[h * GS:(h + 1) * GS], v[:, h * DV:(h + 1) * DV],
        (((1,), (0,)), ((), ())),
        preferred_element_type=jnp.float32) for h in range(HEADS)]
    o = jnp.concatenate(outs, axis=1)
    out_ref[...] = jnp.dot(o, wp_ref[...], preferred_element_type=jnp.float32)


def kernel(x, labels, scores, Wq, Wk, Wv, Wp):
    B, N, _ = x.shape
    ng = N // GS
    x0, lab0, sc0 = x[0], labels[0], scores[0]
    x_ext = jnp.concatenate([x0, x0[N - GS:][::-1]], axis=0)
    lab_ext = jnp.concatenate([lab0, lab0[N - GS:][::-1]], axis=0)
    sc_ext = jnp.concatenate([sc0, sc0[N - GS:][::-1]], axis=0)

    R = 256
    n_ext = N + GS
    n_pad = ((n_ext + R - 1) // R) * R
    x_ext = jnp.pad(x_ext, ((0, n_pad - n_ext), (0, 0)))

    q_ext, k_ext, v_ext = pl.pallas_call(
        _proj_kernel,
        grid=(n_pad // R,),
        in_specs=[
            pl.BlockSpec((R, DIM), lambda i: (i, 0)),
            pl.BlockSpec((DIM, QK_DIM), lambda i: (0, 0)),
            pl.BlockSpec((DIM, QK_DIM), lambda i: (0, 0)),
            pl.BlockSpec((DIM, DIM), lambda i: (0, 0)),
        ],
        out_specs=[
            pl.BlockSpec((R, QK_DIM), lambda i: (i, 0)),
            pl.BlockSpec((R, QK_DIM), lambda i: (i, 0)),
            pl.BlockSpec((R, DIM), lambda i: (i, 0)),
        ],
        out_shape=[
            jax.ShapeDtypeStruct((n_pad, QK_DIM), jnp.float32),
            jax.ShapeDtypeStruct((n_pad, QK_DIM), jnp.float32),
            jax.ShapeDtypeStruct((n_pad, DIM), jnp.float32),
        ],
    )(x_ext, Wq, Wk, Wv)

    lab3 = lab_ext.reshape(n_ext // GS, 1, GS)
    sc3 = sc_ext.reshape(n_ext // GS, 1, GS)

    out = pl.pallas_call(
        _attn_kernel,
        grid=(ng,),
        in_specs=[
            pl.BlockSpec((GS, QK_DIM), lambda i: (i, 0)),
            pl.BlockSpec((GS, QK_DIM), lambda i: (i, 0)),
            pl.BlockSpec((GS, QK_DIM), lambda i: (i + 1, 0)),
            pl.BlockSpec((GS, DIM), lambda i: (i, 0)),
            pl.BlockSpec((GS, DIM), lambda i: (i + 1, 0)),
            pl.BlockSpec((1, 1, GS), lambda i: (i, 0, 0)),
            pl.BlockSpec((1, 1, GS), lambda i: (i, 0, 0)),
            pl.BlockSpec((1, 1, GS), lambda i: (i + 1, 0, 0)),
            pl.BlockSpec((1, 1, GS), lambda i: (i, 0, 0)),
            pl.BlockSpec((1, 1, GS), lambda i: (i, 0, 0)),
            pl.BlockSpec((1, 1, GS), lambda i: (i + 1, 0, 0)),
            pl.BlockSpec((DIM, DIM), lambda i: (0, 0)),
        ],
        out_specs=pl.BlockSpec((GS, DIM), lambda i: (i, 0)),
        out_shape=jax.ShapeDtypeStruct((N, DIM), jnp.float32),
    )(q_ext, k_ext, k_ext, v_ext, v_ext,
      lab3, lab3, lab3, sc3, sc3, sc3, Wp)

    return out[None]


# speedup vs baseline: 20.4097x; 1.5077x over previous
import jax
import jax.numpy as jnp
from jax.experimental import pallas as pl
from jax.experimental.pallas import tpu as pltpu

DIM = 1024
QK_DIM = 1024
HEADS = 16
GS = 64
WS = 2 * GS
NUM_CLUSTERS = 8
R_BASE, R_MIN, R_MAX = 0.5, 0.25, 0.75
LAMBDA_P, LAMBDA_V = 0.25, 0.25
CROSS_RATIO = 0.125
DQK = QK_DIM // HEADS
DV = DIM // HEADS


def _proj_kernel(x_ref, wq_ref, wk_ref, wv_ref, q_ref, k_ref, v_ref):
    x = x_ref[...]
    q_ref[...] = jnp.dot(x, wq_ref[...], preferred_element_type=jnp.float32)
    k_ref[...] = jnp.dot(x, wk_ref[...], preferred_element_type=jnp.float32)
    v_ref[...] = jnp.dot(x, wv_ref[...], preferred_element_type=jnp.float32)


def _attn_kernel(q_ref, k0_ref, k1_ref, v0_ref, v1_ref,
                 ql_ref, kl0_ref, kl1_ref, qs_ref, ks0_ref, ks1_ref,
                 wp_ref, out_ref):
    q = q_ref[...]
    k = jnp.concatenate([k0_ref[...], k1_ref[...]], axis=0)
    v = jnp.concatenate([v0_ref[...], v1_ref[...]], axis=0)
    ql = ql_ref[0]
    kl = jnp.concatenate([kl0_ref[0], kl1_ref[0]], axis=1)
    qs = qs_ref[0]
    ks = jnp.concatenate([ks0_ref[0], ks1_ref[0]], axis=1)

    cl = jax.lax.broadcasted_iota(jnp.int32, (NUM_CLUSTERS, GS), 0)
    counts = jnp.sum((ql == cl).astype(jnp.float32), axis=1, keepdims=True)
    maxc = jnp.max(counts)
    cid = jax.lax.broadcasted_iota(jnp.int32, (NUM_CLUSTERS, 1), 0)
    mode = jnp.min(jnp.where(counts == maxc, cid, NUM_CLUSTERS))
    purity = jnp.mean((ql == mode).astype(jnp.float32))
    smean = jnp.mean(qs)
    svar = jnp.mean((qs - smean) ** 2)
    focus = jnp.clip(R_BASE + LAMBDA_P * purity - LAMBDA_V * svar, R_MIN, R_MAX)
    keep = jnp.clip(jnp.ceil(focus * WS), 1.0, float(WS))
    cross_keep = jnp.round(keep * CROSS_RATIO)
    cross_keep = jnp.where(keep > 1.0, jnp.maximum(cross_keep, 1.0), 0.0)
    same_keep = jnp.maximum(keep - cross_keep, 1.0)

    same_mask = (ql[:, :, None] == kl[:, None, :])[0]
    conf = (qs[:, :, None] * ks[:, None, :])[0]
    lane = jax.lax.broadcasted_iota(jnp.int32, (1, WS), 1)
    n_same = jnp.sum(same_mask.astype(jnp.float32), axis=1, keepdims=True)

    def sort_desc(x):
        for s_exp in range(1, 8):
            s = 1 << s_exp
            for d_exp in range(s_exp - 1, -1, -1):
                d = 1 << d_exp
                has_d = (lane & d) != 0
                idx = jnp.broadcast_to(lane ^ d, x.shape)
                partner = jnp.take_along_axis(x, idx, axis=1)
                take_max = ((lane & s) == 0) ^ has_d
                x = jnp.where(take_max, jnp.maximum(x, partner),
                              jnp.minimum(x, partner))
        return x

    kk2 = jax.lax.broadcasted_iota(jnp.int32, (WS, WS), 0)
    jj2 = jax.lax.broadcasted_iota(jnp.int32, (WS, WS), 1)
    ones_m = jnp.ones((WS, WS), dtype=jnp.bfloat16)
    sl_m = (kk2 < jj2).astype(jnp.bfloat16)

    def rank_mask(enc, srt, kn):
        idx = jnp.minimum(kn, float(WS)).astype(jnp.int32) - 1
        t = jnp.sum(jnp.where(lane == idx, srt, 0.0),
                    axis=1, keepdims=True)
        gt = enc > t
        eq = enc == t
        cnt = (jax.lax.dot_general(
                   gt.astype(jnp.bfloat16), ones_m,
                   (((1,), (0,)), ((), ())),
                   preferred_element_type=jnp.float32) +
               jax.lax.dot_general(
                   eq.astype(jnp.bfloat16), sl_m,
                   (((1,), (0,)), ((), ())),
                   preferred_element_type=jnp.float32))
        return gt | (eq & (cnt < kn))

    scale = DQK ** (-0.5)
    logits = [jax.lax.dot_general(
        q[:, h * DQK:(h + 1) * DQK], k[:, h * DQK:(h + 1) * DQK],
        (((1,), (1,)), ((), ())),
        preferred_element_type=jnp.float32) for h in range(HEADS)]
    lg = jnp.concatenate(logits, axis=0) * scale
    m = jnp.max(lg, axis=1, keepdims=True)
    e = jnp.exp(lg - m)

    same_t = jnp.tile(same_mask, (HEADS, 1))
    conf_t = jnp.tile(conf, (HEADS, 1))
    kn_cross = jnp.tile(n_same + cross_keep, (HEADS, 1))

    enc = jnp.where(same_t, e + 2.0, e * conf_t)
    srt = sort_desc(enc)
    fm = ((rank_mask(enc, srt, same_keep) & same_t) |
          (rank_mask(enc, srt, kn_cross) & ~same_t))
    att = jnp.where(fm, e, 0.0)
    att = att / (jnp.sum(att, axis=1, keepdims=True) + 1e-9)

    outs = [jax.lax.dot_general(
        att[h * GS:(h + 1) * GS], v[:, h * DV:(h + 1) * DV],
        (((1,), (0,)), ((), ())),
        preferred_element_type=jnp.float32) for h in range(HEADS)]
    o = jnp.concatenate(outs, axis=1)
    out_ref[...] = jnp.dot(o, wp_ref[...], preferred_element_type=jnp.float32)


def kernel(x, labels, scores, Wq, Wk, Wv, Wp):
    B, N, _ = x.shape
    ng = N // GS
    x0, lab0, sc0 = x[0], labels[0], scores[0]
    x_ext = jnp.concatenate([x0, x0[N - GS:][::-1]], axis=0)
    lab_ext = jnp.concatenate([lab0, lab0[N - GS:][::-1]], axis=0)
    sc_ext = jnp.concatenate([sc0, sc0[N - GS:][::-1]], axis=0)

    R = 256
    n_ext = N + GS
    n_pad = ((n_ext + R - 1) // R) * R
    x_ext = jnp.pad(x_ext, ((0, n_pad - n_ext), (0, 0)))

    q_ext, k_ext, v_ext = pl.pallas_call(
        _proj_kernel,
        grid=(n_pad // R,),
        in_specs=[
            pl.BlockSpec((R, DIM), lambda i: (i, 0)),
            pl.BlockSpec((DIM, QK_DIM), lambda i: (0, 0)),
            pl.BlockSpec((DIM, QK_DIM), lambda i: (0, 0)),
            pl.BlockSpec((DIM, DIM), lambda i: (0, 0)),
        ],
        out_specs=[
            pl.BlockSpec((R, QK_DIM), lambda i: (i, 0)),
            pl.BlockSpec((R, QK_DIM), lambda i: (i, 0)),
            pl.BlockSpec((R, DIM), lambda i: (i, 0)),
        ],
        out_shape=[
            jax.ShapeDtypeStruct((n_pad, QK_DIM), jnp.float32),
            jax.ShapeDtypeStruct((n_pad, QK_DIM), jnp.float32),
            jax.ShapeDtypeStruct((n_pad, DIM), jnp.float32),
        ],
    )(x_ext, Wq, Wk, Wv)

    lab3 = lab_ext.reshape(n_ext // GS, 1, GS)
    sc3 = sc_ext.reshape(n_ext // GS, 1, GS)

    out = pl.pallas_call(
        _attn_kernel,
        grid=(ng,),
        in_specs=[
            pl.BlockSpec((GS, QK_DIM), lambda i: (i, 0)),
            pl.BlockSpec((GS, QK_DIM), lambda i: (i, 0)),
            pl.BlockSpec((GS, QK_DIM), lambda i: (i + 1, 0)),
            pl.BlockSpec((GS, DIM), lambda i: (i, 0)),
            pl.BlockSpec((GS, DIM), lambda i: (i + 1, 0)),
            pl.BlockSpec((1, 1, GS), lambda i: (i, 0, 0)),
            pl.BlockSpec((1, 1, GS), lambda i: (i, 0, 0)),
            pl.BlockSpec((1, 1, GS), lambda i: (i + 1, 0, 0)),
            pl.BlockSpec((1, 1, GS), lambda i: (i, 0, 0)),
            pl.BlockSpec((1, 1, GS), lambda i: (i, 0, 0)),
            pl.BlockSpec((1, 1, GS), lambda i: (i + 1, 0, 0)),
            pl.BlockSpec((DIM, DIM), lambda i: (0, 0)),
        ],
        out_specs=pl.BlockSpec((GS, DIM), lambda i: (i, 0)),
        out_shape=jax.ShapeDtypeStruct((N, DIM), jnp.float32),
    )(q_ext, k_ext, k_ext, v_ext, v_ext,
      lab3, lab3, lab3, sc3, sc3, sc3, Wp)

    return out[None]
